# baseline (speedup 1.0000x reference)
import jax
import jax.numpy as jnp
from jax.experimental import pallas as pl
from jax.experimental.pallas import tpu as pltpu

F = 64
N = F + 1
HID = 4 * F
KIN = N * F * 3
OUT_SIZE = F * F
RB = 128
NHALF = 3


def _gat_body(hF_ref, hT_ref, WfT_ref, WtT_ref, alF_ref, arF_ref, bF_ref,
              alT_ref, arT_ref, bT_ref, outF_ref, outT_ref):
    def one(h, WT, al, ar, b):
        feat = jnp.dot(h, WT, preferred_element_type=jnp.float32)
        el = feat * al
        er0 = feat[0:1, :] * ar
        e = el + er0
        e = jnp.where(e >= 0.0, e, 0.2 * e)
        m = jnp.max(e, axis=0, keepdims=True)
        w = jnp.exp(e - m)
        s = jnp.sum(w, axis=0, keepdims=True)
        att = jnp.sum(w * feat, axis=0, keepdims=True) / s
        return jnp.concatenate([att, feat[1:, :]], axis=0) + b

    outF_ref[...] = one(hF_ref[...], WfT_ref[...], alF_ref[...], arF_ref[...], bF_ref[...])
    outT_ref[...] = one(hT_ref[...], WtT_ref[...], alT_ref[...], arT_ref[...], bT_ref[...])


def _gru_body(x_ref, WihA_ref, WihB_ref, Whh_ref, bih_ref, bhh_ref, h0_ref,
              out_ref, h1_ref, y_acc, g_acc):
    i = pl.program_id(0)
    x = x_ref[...]
    y_acc[i, :] = jnp.sum(WihA_ref[...] * x, axis=1)
    g_acc[i, :] = jnp.sum(WihB_ref[...] * x, axis=1)

    @pl.when(i == NHALF - 1)
    def _epilogue():
        h0 = h0_ref[...]
        W = Whh_ref[...]
        xr = jnp.concatenate([y_acc[0, :], y_acc[1, :]]) + bih_ref[0, 0:HID]
        xz = jnp.concatenate([y_acc[2, :], g_acc[0, :]]) + bih_ref[0, HID:2 * HID]
        xn = jnp.concatenate([g_acc[1, :], g_acc[2, :]]) + bih_ref[0, 2 * HID:]
        hr = jnp.sum(W[0:HID, :] * h0, axis=1) + bhh_ref[0, 0:HID]
        hz = jnp.sum(W[HID:2 * HID, :] * h0, axis=1) + bhh_ref[0, HID:2 * HID]
        hn = jnp.sum(W[2 * HID:, :] * h0, axis=1) + bhh_ref[0, 2 * HID:]
        r = jax.nn.sigmoid(xr + hr)
        z = jax.nn.sigmoid(xz + hz)
        n = jnp.tanh(xn + r * hn)
        h1 = (1.0 - z) * n + z * h0[0]
        out_ref[...] = jnp.concatenate(
            [h1, jnp.zeros((OUT_SIZE - HID,), jnp.float32)], axis=0)
        h1_ref[0, 0, :] = h1


def kernel(data, hidden, W_feat, al_feat, ar_feat, b_feat,
           W_time, al_time, ar_time, b_time, W_ih, W_hh, b_ih, b_hh):
    f32 = jnp.float32
    z1 = jnp.zeros((1, F), f32)
    hF = jnp.concatenate([z1, data], axis=0)
    hT = jnp.concatenate([z1, data.T], axis=0)

    gat = pl.pallas_call(
        _gat_body,
        out_shape=(jax.ShapeDtypeStruct((N, F), f32),
                   jax.ShapeDtypeStruct((N, F), f32)),
    )
    fRF, fRT = gat(hF, hT, W_feat.T, W_time.T,
                   al_feat.reshape(1, F), ar_feat.reshape(1, F), b_feat.reshape(1, F),
                   al_time.reshape(1, F), ar_time.reshape(1, F), b_time.reshape(1, F))

    x = jnp.stack([hF, fRF, fRT], axis=-1).reshape(1, KIN)

    full = lambda shape: pl.BlockSpec(shape, lambda i: tuple(0 for _ in shape))
    gru = pl.pallas_call(
        _gru_body,
        grid=(NHALF,),
        in_specs=[
            full((1, KIN)),
            pl.BlockSpec((RB, KIN), lambda i: (i, 0)),
            pl.BlockSpec((RB, KIN), lambda i: (i + NHALF, 0)),
            full((3 * HID, HID)),
            full((1, 3 * HID)), full((1, 3 * HID)),
            full((1, HID)),
        ],
        out_specs=(full((OUT_SIZE,)), full((1, 1, HID))),
        out_shape=(jax.ShapeDtypeStruct((OUT_SIZE,), f32),
                   jax.ShapeDtypeStruct((1, 1, HID), f32)),
        scratch_shapes=[pltpu.VMEM((NHALF, RB), f32),
                        pltpu.VMEM((NHALF, RB), f32)],
    )
    out, h1 = gru(x, W_ih, W_ih, W_hh, b_ih.reshape(1, 3 * HID),
                  b_hh.reshape(1, 3 * HID), hidden.reshape(1, HID))
    return out, h1


# speedup vs baseline: 1.0097x; 1.0097x over previous
import jax
import jax.numpy as jnp
from jax.experimental import pallas as pl
from jax.experimental.pallas import tpu as pltpu

F = 64
N = F + 1
HID = 4 * F
KIN = N * F * 3
OUT_SIZE = F * F
NCHUNK = 16
CR = (3 * HID) // NCHUNK


def _gat_body(hF_ref, hT_ref, WfT_ref, WtT_ref, alF_ref, arF_ref, bF_ref,
              alT_ref, arT_ref, bT_ref, outF_ref, outT_ref):
    def one(h, WT, al, ar, b):
        feat = jnp.dot(h, WT, preferred_element_type=jnp.float32)
        el = feat * al
        er0 = feat[0:1, :] * ar
        e = el + er0
        e = jnp.where(e >= 0.0, e, 0.2 * e)
        m = jnp.max(e, axis=0, keepdims=True)
        w = jnp.exp(e - m)
        s = jnp.sum(w, axis=0, keepdims=True)
        att = jnp.sum(w * feat, axis=0, keepdims=True) / s
        return jnp.concatenate([att, feat[1:, :]], axis=0) + b

    outF_ref[...] = one(hF_ref[...], WfT_ref[...], alF_ref[...], arF_ref[...], bF_ref[...])
    outT_ref[...] = one(hT_ref[...], WtT_ref[...], alT_ref[...], arT_ref[...], bT_ref[...])


def _gru_body(x_ref, Whbm_ref, Whh_ref, bih_ref, bhh_ref, h0_ref,
              out_ref, h1_ref, wbuf, y_scr, sems):
    copies = [
        pltpu.make_async_copy(
            Whbm_ref.at[pl.ds(c * CR, CR), :],
            wbuf.at[pl.ds(c * CR, CR), :],
            sems.at[c])
        for c in range(NCHUNK)
    ]
    for cp in copies:
        cp.start()

    x = x_ref[...]
    for c in range(NCHUNK):
        copies[c].wait()
        w = wbuf[pl.ds(c * CR, CR), :]
        y_scr[0, c * CR:(c + 1) * CR] = jnp.sum(w * x, axis=1)

    h0 = h0_ref[...]
    W = Whh_ref[...]
    gx = y_scr[0, :] + bih_ref[0, :]
    xr, xz, xn = gx[0:HID], gx[HID:2 * HID], gx[2 * HID:]
    hr = jnp.sum(W[0:HID, :] * h0, axis=1) + bhh_ref[0, 0:HID]
    hz = jnp.sum(W[HID:2 * HID, :] * h0, axis=1) + bhh_ref[0, HID:2 * HID]
    hn = jnp.sum(W[2 * HID:, :] * h0, axis=1) + bhh_ref[0, 2 * HID:]
    r = jax.nn.sigmoid(xr + hr)
    z = jax.nn.sigmoid(xz + hz)
    n = jnp.tanh(xn + r * hn)
    h1 = (1.0 - z) * n + z * h0[0]
    out_ref[...] = jnp.concatenate(
        [h1, jnp.zeros((OUT_SIZE - HID,), jnp.float32)], axis=0)
    h1_ref[0, 0, :] = h1


def kernel(data, hidden, W_feat, al_feat, ar_feat, b_feat,
           W_time, al_time, ar_time, b_time, W_ih, W_hh, b_ih, b_hh):
    f32 = jnp.float32
    z1 = jnp.zeros((1, F), f32)
    hF = jnp.concatenate([z1, data], axis=0)
    hT = jnp.concatenate([z1, data.T], axis=0)

    gat = pl.pallas_call(
        _gat_body,
        out_shape=(jax.ShapeDtypeStruct((N, F), f32),
                   jax.ShapeDtypeStruct((N, F), f32)),
    )
    fRF, fRT = gat(hF, hT, W_feat.T, W_time.T,
                   al_feat.reshape(1, F), ar_feat.reshape(1, F), b_feat.reshape(1, F),
                   al_time.reshape(1, F), ar_time.reshape(1, F), b_time.reshape(1, F))

    x = jnp.stack([hF, fRF, fRT], axis=-1).reshape(1, KIN)

    gru = pl.pallas_call(
        _gru_body,
        in_specs=[
            pl.BlockSpec(memory_space=pltpu.MemorySpace.VMEM),
            pl.BlockSpec(memory_space=pltpu.MemorySpace.HBM),
            pl.BlockSpec(memory_space=pltpu.MemorySpace.VMEM),
            pl.BlockSpec(memory_space=pltpu.MemorySpace.VMEM),
            pl.BlockSpec(memory_space=pltpu.MemorySpace.VMEM),
            pl.BlockSpec(memory_space=pltpu.MemorySpace.VMEM),
        ],
        out_specs=(pl.BlockSpec(memory_space=pltpu.MemorySpace.VMEM),
                   pl.BlockSpec(memory_space=pltpu.MemorySpace.VMEM)),
        out_shape=(jax.ShapeDtypeStruct((OUT_SIZE,), f32),
                   jax.ShapeDtypeStruct((1, 1, HID), f32)),
        scratch_shapes=[pltpu.VMEM((3 * HID, KIN), f32),
                        pltpu.VMEM((1, 3 * HID), f32),
                        pltpu.SemaphoreType.DMA((NCHUNK,))],
    )
    out, h1 = gru(x, W_ih, W_hh, b_ih.reshape(1, 3 * HID),
                  b_hh.reshape(1, 3 * HID), hidden.reshape(1, HID))
    return out, h1
